# Initial kernel scaffold; baseline (speedup 1.0000x reference)
import jax
import jax.numpy as jnp
from jax.experimental import pallas as pl


def kernel(x, adj, useless, W1, b1, g1, be1, W2, b2, g2, be2, W3, b3, g3, be3):
    raise NotImplementedError("write your pallas kernel here")


# speedup vs baseline: 3.5227x; 3.5227x over previous
import jax
import jax.numpy as jnp
from jax.experimental import pallas as pl

N = 8192
D = 64
BM = 256
NB = N // BM
EPS = 1e-5


def _prep_kernel(adj_ref, asl_ref, abf_ref):
    i = pl.program_id(0)

    dblk = adj_ref[:, pl.ds(i * BM, BM)]
    eye = (jax.lax.broadcasted_iota(jnp.int32, (BM, BM), 0)
           == jax.lax.broadcasted_iota(jnp.int32, (BM, BM), 1))
    adj_ref[:, pl.ds(i * BM, BM)] = jnp.where(eye, 1.0, dblk)

    a = adj_ref[...]
    asl_ref[...] = a
    abf_ref[...] = a.astype(jnp.bfloat16)


def _layer_kernel(z_ref, abf_ref, b_ref, out_ref):
    a = abf_ref[...].astype(jnp.float32)
    o = jnp.dot(a, z_ref[...], preferred_element_type=jnp.float32)
    out_ref[...] = jnp.maximum(o + b_ref[...], 0.0)


def _bn(h, g, b):
    m = jnp.mean(h, axis=0)
    v = jnp.var(h, axis=0)
    return g * (h - m) / jnp.sqrt(v + EPS) + b


def kernel(x, adj, useless, W1, b1, g1, be1, W2, b2, g2, be2, W3, b3, g3, be3):
    adj_sl, adj_bf = pl.pallas_call(
        _prep_kernel,
        grid=(NB,),
        in_specs=[pl.BlockSpec((BM, N), lambda i: (i, 0))],
        out_specs=[
            pl.BlockSpec((BM, N), lambda i: (i, 0)),
            pl.BlockSpec((BM, N), lambda i: (i, 0)),
        ],
        out_shape=[
            jax.ShapeDtypeStruct((N, N), jnp.float32),
            jax.ShapeDtypeStruct((N, N), jnp.bfloat16),
        ],
    )(adj)

    out1 = adj_sl @ (x @ W1) + b1
    out1 = jax.nn.relu(out1)
    h1 = _bn(out1, g1, be1)

    def _layer(h, W, b):
        z = jnp.dot(h, W)
        return pl.pallas_call(
            _layer_kernel,
            grid=(NB,),
            in_specs=[
                pl.BlockSpec((N, D), lambda i: (0, 0)),
                pl.BlockSpec((BM, N), lambda i: (i, 0)),
                pl.BlockSpec((1, D), lambda i: (0, 0)),
            ],
            out_specs=pl.BlockSpec((BM, D), lambda i: (i, 0)),
            out_shape=jax.ShapeDtypeStruct((N, D), jnp.float32),
        )(z, adj_bf, b.reshape(1, D))

    h2 = _bn(_layer(h1, W2, b2), g2, be2)
    h3 = _bn(_layer(h2, W3, b3), g3, be3)
    return h3
